# baseline (speedup 1.0000x reference)
import functools

import jax
import jax.numpy as jnp
from jax import lax
from jax.experimental import pallas as pl
from jax.experimental.pallas import tpu as pltpu
from jax.experimental.pallas import tpu_sc as plsc

NTOKEN = 1000000
EMB_DIM = 32
BATCH = 16384
HIST = 200

NC = 2
NS = 16
NW = NC * NS

CHUNK = 512
NBT = CHUNK // 128
N_CHUNKS = BATCH // CHUNK
XP = CHUNK * EMB_DIM
H_STRIDE = BATCH * EMB_DIM
DT_STRIDE = 8 * BATCH
OUT_ELEMS = BATCH * HIST * EMB_DIM


def _make_kernel():
    mesh = plsc.VectorSubcoreMesh(core_axis_name="c", subcore_axis_name="s")

    @functools.partial(
        pl.kernel,
        mesh=mesh,
        out_type=jax.ShapeDtypeStruct((OUT_ELEMS,), jnp.float32),
        scratch_types=[
            pltpu.VMEM((BATCH,), jnp.int32),
            pltpu.VMEM((CHUNK, EMB_DIM), jnp.float32),
            pltpu.VMEM((CHUNK, EMB_DIM), jnp.float32),
            pltpu.VMEM((XP,), jnp.float32),
            pltpu.VMEM((XP,), jnp.float32),
            pltpu.SemaphoreType.DMA,
            pltpu.SemaphoreType.DMA,
            pltpu.SemaphoreType.DMA,
            pltpu.SemaphoreType.DMA,
        ],
        compiler_params=pltpu.CompilerParams(
            use_tc_tiling_on_sc=False, needs_layout_passes=False),
    )
    def emb_kernel(idx_hbm, table_hbm, out_hbm, idx_v, rows0, rows1,
                   xp0, xp1, sem_g0, sem_g1, sem_s0, sem_s1):
        wid = lax.axis_index("s") * NC + lax.axis_index("c")
        n_h = jnp.where(wid < HIST % NW, HIST // NW + 1, HIST // NW)
        rows = (rows0, rows1)
        xp = (xp0, xp1)
        sg = (sem_g0, sem_g1)
        ss = (sem_s0, sem_s1)
        iota16 = lax.iota(jnp.int32, 16)

        def start_gather(c, k):
            return pltpu.async_copy(
                table_hbm.at[idx_v.at[pl.ds(c * CHUNK, CHUNK)]], rows[k], sg[k])

        def wait_gather(k):
            pltpu.make_async_copy(
                table_hbm.at[idx_v.at[pl.ds(0, CHUNK)]], rows[k], sg[k]).wait()

        SEG = NBT * 1024

        def start_stores(c, k, h_base):
            for dt in range(4):
                pltpu.async_copy(
                    xp[k].at[pl.ds(dt * SEG, SEG)],
                    out_hbm.at[pl.ds(h_base + dt * DT_STRIDE + c * SEG, SEG)],
                    ss[k])

        def wait_stores(k):
            for dt in range(4):
                pltpu.make_async_copy(
                    xp[k].at[pl.ds(dt * SEG, SEG)],
                    out_hbm.at[pl.ds(dt * SEG, SEG)],
                    ss[k]).wait()

        def transpose(k):
            def bt_body(bt, carry):
                bto = bt * 1024
                ridx = [bt * 128 + kk * 16 + iota16 for kk in range(8)]
                for dt in range(4):
                    for dr in range(8):
                        col = jnp.full((16,), dt * 8 + dr, jnp.int32)
                        off = dt * (NBT * 1024) + dr * 128
                        for kk in range(8):
                            v = plsc.load_gather(rows[k], [ridx[kk], col])
                            xp[k][pl.ds(bto + off + kk * 16, 16)] = v
                return carry

            lax.fori_loop(0, NBT, bt_body, 0)

        def h_body(ih, carry):
            h = wid + NW * ih
            pltpu.sync_copy(idx_hbm.at[pl.ds(h * BATCH, BATCH)], idx_v)
            h_base = h * H_STRIDE
            start_gather(0, 0)

            def chunk_body(si, carry2):
                for kk in range(2):
                    c = si * 2 + kk
                    wait_gather(kk)

                    @pl.when(jnp.logical_or(si < N_CHUNKS // 2 - 1, kk == 0))
                    def _():
                        start_gather(c + 1, 1 - kk)

                    @pl.when(si > 0)
                    def _():
                        wait_stores(kk)

                    transpose(kk)
                    start_stores(c, kk, h_base)
                return carry2

            lax.fori_loop(0, N_CHUNKS // 2, chunk_body, 0)
            wait_stores(0)
            wait_stores(1)
            return carry

        lax.fori_loop(0, n_h, h_body, 0)

    return emb_kernel


_emb_kernel = _make_kernel()


@jax.jit
def kernel(x, emb_weight):
    idx_hs = x.T.reshape(-1).astype(jnp.int32)
    out_flat = _emb_kernel(idx_hs, emb_weight)
    out5 = out_flat.reshape(HIST, 4, 128, 8, 128)
    return jnp.transpose(out5, (2, 4, 0, 1, 3)).reshape(BATCH, HIST, EMB_DIM)


# speedup vs baseline: 3.6645x; 3.6645x over previous
import functools

import jax
import jax.numpy as jnp
from jax import lax
from jax.experimental import pallas as pl
from jax.experimental.pallas import tpu as pltpu
from jax.experimental.pallas import tpu_sc as plsc

NTOKEN = 1000000
EMB_DIM = 32
BATCH = 16384
HIST = 200

NC = 2
NS = 16
NW = NC * NS

CHUNK = 512
NBT = CHUNK // 128
N_CHUNKS = BATCH // CHUNK
XP = CHUNK * EMB_DIM
H_STRIDE = BATCH * EMB_DIM
DT_STRIDE = 8 * BATCH
OUT_ELEMS = BATCH * HIST * EMB_DIM


def _make_kernel():
    mesh = plsc.VectorSubcoreMesh(core_axis_name="c", subcore_axis_name="s")

    @functools.partial(
        pl.kernel,
        mesh=mesh,
        out_type=jax.ShapeDtypeStruct((OUT_ELEMS,), jnp.float32),
        scratch_types=[
            pltpu.VMEM((BATCH,), jnp.int32),
            pltpu.VMEM((CHUNK, EMB_DIM), jnp.float32),
            pltpu.VMEM((CHUNK, EMB_DIM), jnp.float32),
            pltpu.VMEM((XP,), jnp.float32),
            pltpu.VMEM((XP,), jnp.float32),
            pltpu.SemaphoreType.DMA,
            pltpu.SemaphoreType.DMA,
            pltpu.SemaphoreType.DMA,
            pltpu.SemaphoreType.DMA,
        ],
        compiler_params=pltpu.CompilerParams(
            use_tc_tiling_on_sc=False, needs_layout_passes=False),
    )
    def emb_kernel(idx_hbm, table_hbm, out_hbm, idx_v, rows0, rows1,
                   xp0, xp1, sem_g0, sem_g1, sem_s0, sem_s1):
        wid = lax.axis_index("s") * NC + lax.axis_index("c")
        n_h = jnp.where(wid < HIST % NW, HIST // NW + 1, HIST // NW)
        rows = (rows0, rows1)
        xp = (xp0, xp1)
        sg = (sem_g0, sem_g1)
        ss = (sem_s0, sem_s1)
        iota16 = lax.iota(jnp.int32, 16)

        def start_gather(c, k):
            return pltpu.async_copy(
                table_hbm.at[idx_v.at[pl.ds(c * CHUNK, CHUNK)]], rows[k], sg[k])

        def wait_gather(k):
            pltpu.make_async_copy(
                table_hbm.at[idx_v.at[pl.ds(0, CHUNK)]], rows[k], sg[k]).wait()

        SEG = NBT * 1024

        def start_stores(c, k, h_base):
            for dt in range(4):
                pltpu.async_copy(
                    xp[k].at[pl.ds(dt * SEG, SEG)],
                    out_hbm.at[pl.ds(h_base + dt * DT_STRIDE + c * SEG, SEG)],
                    ss[k])

        def wait_stores(k):
            for dt in range(4):
                pltpu.make_async_copy(
                    xp[k].at[pl.ds(dt * SEG, SEG)],
                    out_hbm.at[pl.ds(dt * SEG, SEG)],
                    ss[k]).wait()

        diag = [(iota16 + j) & 15 for j in range(16)]
        cols = [[d + (n * 16) for d in diag] for n in range(2)]
        wpat = [((diag[j] >> 3) << 12) + ((diag[j] & 7) << 7) + iota16
                for j in range(16)]

        def transpose(k):
            def i_body(i, carry):
                rowv = i * 16 + iota16
                base_w = (i // 8) * 1024 + (i % 8) * 16
                for n in range(2):
                    wb = jnp.broadcast_to(base_w + n * 8192, (16,))
                    vs = [plsc.load_gather(rows[k], [rowv, cols[n][j]])
                          for j in range(16)]
                    for j in range(16):
                        plsc.store_scatter(xp[k], [wb + wpat[j]], vs[j])
                return carry

            lax.fori_loop(0, CHUNK // 16, i_body, 0)

        def h_body(ih, carry):
            h = wid + NW * ih
            pltpu.sync_copy(idx_hbm.at[pl.ds(h * BATCH, BATCH)], idx_v)
            h_base = h * H_STRIDE
            start_gather(0, 0)

            def chunk_body(si, carry2):
                for kk in range(2):
                    c = si * 2 + kk
                    wait_gather(kk)

                    @pl.when(jnp.logical_or(si < N_CHUNKS // 2 - 1, kk == 0))
                    def _():
                        start_gather(c + 1, 1 - kk)

                    @pl.when(si > 0)
                    def _():
                        wait_stores(kk)

                    transpose(kk)
                    start_stores(c, kk, h_base)
                return carry2

            lax.fori_loop(0, N_CHUNKS // 2, chunk_body, 0)
            wait_stores(0)
            wait_stores(1)
            return carry

        lax.fori_loop(0, n_h, h_body, 0)

    return emb_kernel


_emb_kernel = _make_kernel()


@jax.jit
def kernel(x, emb_weight):
    idx_hs = x.T.reshape(-1).astype(jnp.int32)
    out_flat = _emb_kernel(idx_hs, emb_weight)
    out5 = out_flat.reshape(HIST, 4, 128, 8, 128)
    return jnp.transpose(out5, (2, 4, 0, 1, 3)).reshape(BATCH, HIST, EMB_DIM)
